# baseline (speedup 1.0000x reference)
import jax
import jax.numpy as jnp
from jax import lax
from jax.experimental import pallas as pl
from jax.experimental.pallas import tpu as pltpu
from jax.experimental.pallas import tpu_sc as plsc

T = 1000
LATENT_DIM = 128
BATCH = 16384

_info = plsc.get_sparse_core_info()
_NC, _NS = _info.num_cores, _info.num_subcores
_NW = _NC * _NS
_CHUNK = 128
_ROWS_PER_W = BATCH // _NW
_CHUNKS_PER_W = _ROWS_PER_W // _CHUNK


def _gather_body(t_hbm, table_hbm, out_hbm, idx_v, rows_v, isems, gsem):
    wid = lax.axis_index("s") * _NC + lax.axis_index("c")
    base = wid * _ROWS_PER_W
    icopies = []
    for j in range(_CHUNKS_PER_W):
        icopies.append(
            pltpu.async_copy(
                t_hbm.at[pl.ds(base + j * _CHUNK, _CHUNK)],
                idx_v.at[pl.ds(j * _CHUNK, _CHUNK)],
                isems.at[j],
            )
        )
    gathers = []
    for j in range(_CHUNKS_PER_W):
        icopies[j].wait()
        gathers.append(
            pltpu.async_copy(
                table_hbm.at[idx_v.at[pl.ds(j * _CHUNK, _CHUNK)]],
                rows_v.at[pl.ds(j * _CHUNK, _CHUNK)],
                gsem,
            )
        )
    for d in gathers:
        d.wait()
    pltpu.sync_copy(rows_v, out_hbm.at[pl.ds(base, _ROWS_PER_W)])


def kernel(t, table):
    mesh = plsc.VectorSubcoreMesh(core_axis_name="c", subcore_axis_name="s")
    return pl.kernel(
        _gather_body,
        out_type=jax.ShapeDtypeStruct((BATCH, LATENT_DIM), jnp.float32),
        mesh=mesh,
        scratch_types=[
            pltpu.VMEM((_ROWS_PER_W,), jnp.int32),
            pltpu.VMEM((_ROWS_PER_W, LATENT_DIM), jnp.float32),
            pltpu.SemaphoreType.DMA((_CHUNKS_PER_W,)),
            pltpu.SemaphoreType.DMA,
        ],
    )(t, table)


# speedup vs baseline: 1.0134x; 1.0134x over previous
import jax
import jax.numpy as jnp
from jax import lax
from jax.experimental import pallas as pl
from jax.experimental.pallas import tpu as pltpu
from jax.experimental.pallas import tpu_sc as plsc

T = 1000
LATENT_DIM = 128
BATCH = 16384

_info = plsc.get_sparse_core_info()
_NC, _NS = _info.num_cores, _info.num_subcores
_NW = _NC * _NS
_CHUNK = 128
_ROWS_PER_W = BATCH // _NW
_CHUNKS_PER_W = _ROWS_PER_W // _CHUNK


def _gather_body(t_hbm, table_hbm, out_hbm, idx_v, rows_v, sem):
    wid = lax.axis_index("s") * _NC + lax.axis_index("c")
    base = wid * _ROWS_PER_W
    pltpu.sync_copy(t_hbm.at[pl.ds(base, _ROWS_PER_W)], idx_v)
    descs = []
    for j in range(_CHUNKS_PER_W):
        descs.append(
            pltpu.async_copy(
                table_hbm.at[idx_v.at[pl.ds(j * _CHUNK, _CHUNK)]],
                rows_v.at[pl.ds(j * _CHUNK, _CHUNK)],
                sem,
            )
        )
    for d in descs:
        d.wait()
    pltpu.sync_copy(rows_v, out_hbm.at[pl.ds(base, _ROWS_PER_W)])


def kernel(t, table):
    mesh = plsc.VectorSubcoreMesh(core_axis_name="c", subcore_axis_name="s")
    return pl.kernel(
        _gather_body,
        out_type=jax.ShapeDtypeStruct((BATCH, LATENT_DIM), jnp.float32),
        mesh=mesh,
        scratch_types=[
            pltpu.VMEM((_ROWS_PER_W,), jnp.int32),
            pltpu.VMEM((_ROWS_PER_W, LATENT_DIM), jnp.float32),
            pltpu.SemaphoreType.DMA,
        ],
    )(t, table)
